# Initial kernel scaffold; baseline (speedup 1.0000x reference)
import jax
import jax.numpy as jnp
from jax.experimental import pallas as pl


def kernel(x, edge_index, W0, b0, W1, b1):
    raise NotImplementedError("write your pallas kernel here")


# speedup vs baseline: 22.0933x; 22.0933x over previous
import jax
import jax.numpy as jnp
from jax import lax
from jax.experimental import pallas as pl
from jax.experimental.pallas import tpu as pltpu
from jax.experimental.pallas import tpu_sc as plsc

N = 10000
E = 320000
D_IN = 128
D_H = 32
D_OUT = 128
D = 128

NC = 2
NS = 16
K = 125
NCHUNK = E // (NC * NS * K)
N_PAD = 10240
DPT = N_PAD // NS

N_HALF = N_PAD // 2
H_PAD = N_HALF + 128
HPT = H_PAD // NS
OPT = N_HALF // NS

_MESH = dict(core_axis_name="c", subcore_axis_name="s")


def _sc_deg_body(dst_hbm, ones_hbm, zer_hbm, out_hbm, didx, ones_v, dacc,
                 s0, s1, s2, s3):
    c = lax.axis_index("c")
    s = lax.axis_index("s")
    pltpu.sync_copy(dst_hbm.at[c, s], didx)
    pltpu.sync_copy(ones_hbm, ones_v)
    pltpu.sync_copy(zer_hbm, dacc.at[pl.ds(s * DPT, DPT)])
    plsc.subcore_barrier()

    sems = [s0, s1, s2, s3]
    for p in range(4):
        pltpu.async_copy(ones_v, dacc.at[didx.at[p]], sems[p], add=True)

    def step(t, carry):
        for p in range(4):
            jj = 4 * t + p
            pltpu.make_async_copy(ones_v, dacc.at[didx.at[jj - 4]],
                                  sems[p]).wait()
            pltpu.async_copy(ones_v, dacc.at[didx.at[jj]], sems[p], add=True)
        return carry

    lax.fori_loop(1, NCHUNK // 4, step, 0)
    for p in range(4):
        jj = NCHUNK - 4 + p
        pltpu.make_async_copy(ones_v, dacc.at[didx.at[jj]], sems[p]).wait()

    plsc.subcore_barrier()
    pltpu.sync_copy(dacc.at[pl.ds(s * DPT, DPT)],
                    out_hbm.at[c, pl.ds(s * DPT, DPT)])


def _sc_deg(dst4, ones_k, z_deg):
    return pl.kernel(
        _sc_deg_body,
        out_type=jax.ShapeDtypeStruct((NC, N_PAD), jnp.float32),
        mesh=plsc.VectorSubcoreMesh(**_MESH),
        scratch_types=[
            pltpu.VMEM((NCHUNK, K), jnp.int32),
            pltpu.VMEM((K,), jnp.float32),
            pltpu.VMEM_SHARED((N_PAD,), jnp.float32),
            pltpu.SemaphoreType.DMA,
            pltpu.SemaphoreType.DMA,
            pltpu.SemaphoreType.DMA,
            pltpu.SemaphoreType.DMA,
        ],
        name="sc_gcn_deg",
    )(dst4, ones_k, z_deg)


def _sc_scatter_body(y_hbm, src_hbm, dstA_hbm, dstB_hbm, zer_hbm, out_hbm,
                     sidx, didx, b0, b1, b2, b3, acc, g0, g1, g2, g3,
                     s0, s1, s2, s3):
    c = lax.axis_index("c")
    s = lax.axis_index("s")
    pltpu.sync_copy(src_hbm.at[c, s], sidx)

    bufs = [b0, b1, b2, b3]
    gsems = [g0, g1, g2, g3]
    ssems = [s0, s1, s2, s3]

    for h, d_hbm in enumerate((dstA_hbm, dstB_hbm)):
        pltpu.sync_copy(d_hbm.at[c, s], didx)
        pltpu.sync_copy(zer_hbm, acc.at[pl.ds(s * HPT, HPT), :])
        plsc.subcore_barrier()

        for p in range(4):
            pltpu.async_copy(y_hbm.at[sidx.at[p]], bufs[p], gsems[p])

        def step(t, carry):
            for p in range(4):
                jj = 4 * t + p
                pltpu.make_async_copy(y_hbm.at[sidx.at[jj]], bufs[p],
                                      gsems[p]).wait()
                pltpu.async_copy(bufs[p], acc.at[didx.at[jj]], ssems[p],
                                 add=True)
                pltpu.make_async_copy(bufs[p], acc.at[didx.at[jj]],
                                      ssems[p]).wait()

                @pl.when(jj + 4 < NCHUNK)
                def _():
                    pltpu.async_copy(y_hbm.at[sidx.at[jj + 4]], bufs[p],
                                     gsems[p])
            return carry

        lax.fori_loop(0, NCHUNK // 4, step, 0)
        plsc.subcore_barrier()
        pltpu.sync_copy(acc.at[pl.ds(s * OPT, OPT), :],
                        out_hbm.at[c, pl.ds(h * N_HALF + s * OPT, OPT), :])
        plsc.subcore_barrier()


def _sc_scatter(y, src4, dstA4, dstB4, zrows):
    return pl.kernel(
        _sc_scatter_body,
        out_type=jax.ShapeDtypeStruct((NC, N_PAD, D), jnp.float32),
        mesh=plsc.VectorSubcoreMesh(**_MESH),
        scratch_types=[
            pltpu.VMEM((NCHUNK, K), jnp.int32),
            pltpu.VMEM((NCHUNK, K), jnp.int32),
            pltpu.VMEM((K, D), jnp.float32),
            pltpu.VMEM((K, D), jnp.float32),
            pltpu.VMEM((K, D), jnp.float32),
            pltpu.VMEM((K, D), jnp.float32),
            pltpu.VMEM_SHARED((H_PAD, D), jnp.float32),
            pltpu.SemaphoreType.DMA,
            pltpu.SemaphoreType.DMA,
            pltpu.SemaphoreType.DMA,
            pltpu.SemaphoreType.DMA,
            pltpu.SemaphoreType.DMA,
            pltpu.SemaphoreType.DMA,
            pltpu.SemaphoreType.DMA,
            pltpu.SemaphoreType.DMA,
        ],
        name="sc_gcn_scatter",
    )(y, src4, dstA4, dstB4, zrows)


R = 1000


def _tc_prep_body(x_ref, d0_ref, d1_ref, g_ref, dinv_ref):
    dinv = lax.rsqrt(d0_ref[...] + d1_ref[...] + 1.0)
    g_ref[...] = x_ref[...] * dinv
    dinv_ref[...] = dinv


def _tc_prep(x, d0, d1):
    return pl.pallas_call(
        _tc_prep_body,
        grid=(N // R,),
        in_specs=[
            pl.BlockSpec((R, D_IN), lambda i: (i, 0)),
            pl.BlockSpec((R, 1), lambda i: (i, 0)),
            pl.BlockSpec((R, 1), lambda i: (i, 0)),
        ],
        out_specs=[
            pl.BlockSpec((R, D_IN), lambda i: (i, 0)),
            pl.BlockSpec((R, 1), lambda i: (i, 0)),
        ],
        out_shape=[
            jax.ShapeDtypeStruct((N, D_IN), jnp.float32),
            jax.ShapeDtypeStruct((N, 1), jnp.float32),
        ],
    )(x, d0, d1)


def _tc_mid_body(acc_ref, g_ref, dinv_ref, b0_ref, w0_ref, w1_ref, y1_ref):
    dinv = dinv_ref[...]
    agg = acc_ref[0] + acc_ref[1] + g_ref[...]
    pre = jnp.dot(agg, w0_ref[...],
                  preferred_element_type=jnp.float32) * dinv + b0_ref[...]
    h = jnp.maximum(pre, 0.0)
    y1_ref[...] = jnp.dot(h, w1_ref[...],
                          preferred_element_type=jnp.float32) * dinv


def _tc_mid(accp, g, dinv, b0, w0, w1):
    return pl.pallas_call(
        _tc_mid_body,
        grid=(N // R,),
        in_specs=[
            pl.BlockSpec((NC, R, D_IN), lambda i: (0, i, 0)),
            pl.BlockSpec((R, D_IN), lambda i: (i, 0)),
            pl.BlockSpec((R, 1), lambda i: (i, 0)),
            pl.BlockSpec((1, D_H), lambda i: (0, 0)),
            pl.BlockSpec((D_IN, D_H), lambda i: (0, 0)),
            pl.BlockSpec((D_H, D_OUT), lambda i: (0, 0)),
        ],
        out_specs=pl.BlockSpec((R, D_OUT), lambda i: (i, 0)),
        out_shape=jax.ShapeDtypeStruct((N, D_OUT), jnp.float32),
    )(accp, g, dinv, b0, w0, w1)


def _tc_final_body(acc_ref, y1_ref, dinv_ref, b1_ref, out_ref):
    pre = (acc_ref[0] + acc_ref[1] + y1_ref[...]) * dinv_ref[...] + b1_ref[...]
    out_ref[...] = jnp.maximum(pre, 0.0)


def _tc_final(accp, y1, dinv, b1):
    return pl.pallas_call(
        _tc_final_body,
        grid=(N // R,),
        in_specs=[
            pl.BlockSpec((NC, R, D_OUT), lambda i: (0, i, 0)),
            pl.BlockSpec((R, D_OUT), lambda i: (i, 0)),
            pl.BlockSpec((R, 1), lambda i: (i, 0)),
            pl.BlockSpec((1, D_OUT), lambda i: (0, 0)),
        ],
        out_specs=pl.BlockSpec((R, D_OUT), lambda i: (i, 0)),
        out_shape=jax.ShapeDtypeStruct((N, D_OUT), jnp.float32),
    )(accp, y1, dinv, b1)


@jax.jit
def kernel(x, edge_index, W0, b0, W1, b1):
    src = edge_index[0].astype(jnp.int32)
    dst = edge_index[1].astype(jnp.int32)
    src4 = src.reshape(NC, NS, NCHUNK, K)
    dst4 = dst.reshape(NC, NS, NCHUNK, K)
    dump = N_HALF + (dst & 127)
    dstA4 = jnp.where(dst < N_HALF, dst, dump).reshape(NC, NS, NCHUNK, K)
    dstB4 = jnp.where(dst >= N_HALF, dst - N_HALF,
                      dump).reshape(NC, NS, NCHUNK, K)
    ones_k = jnp.ones((K,), jnp.float32)
    z_deg = jnp.zeros((DPT,), jnp.float32)
    zrows = jnp.zeros((HPT, D), jnp.float32)

    degp = _sc_deg(dst4, ones_k, z_deg)
    d0 = degp[0, :N].reshape(N, 1)
    d1 = degp[1, :N].reshape(N, 1)

    g, dinv = _tc_prep(x, d0, d1)
    accp0 = _sc_scatter(g, src4, dstA4, dstB4, zrows)
    y1 = _tc_mid(accp0, g, dinv, b0.reshape(1, D_H), W0, W1)
    accp1 = _sc_scatter(y1, src4, dstA4, dstB4, zrows)
    return _tc_final(accp1, y1, dinv, b1.reshape(1, D_OUT))
